# baseline (speedup 1.0000x reference)
import functools

import jax
import jax.numpy as jnp
from jax.experimental import pallas as pl

N = 20000
INPUT_DIM = 1024
ROW_BLOCK = 2000


def _fused_heads_kernel(x_ref, wc_ref, bc_ref, wb_ref, bb_ref,
                        scores_ref, deltas_ref):
    x = x_ref[...]
    dn = (((1,), (1,)), ((), ()))
    scores_ref[...] = (
        jax.lax.dot_general(x, wc_ref[...], dn,
                            preferred_element_type=jnp.float32)
        + bc_ref[...][None, :]
    )
    deltas_ref[...] = (
        jax.lax.dot_general(x, wb_ref[...], dn,
                            preferred_element_type=jnp.float32)
        + bb_ref[...][None, :]
    )


@jax.jit
def kernel(x, W_cls, b_cls, W_bbox, b_bbox):
    n_cls = W_cls.shape[0]
    n_box = W_bbox.shape[0]
    grid = (N // ROW_BLOCK,)
    scores, deltas = pl.pallas_call(
        _fused_heads_kernel,
        grid=grid,
        in_specs=[
            pl.BlockSpec((ROW_BLOCK, INPUT_DIM), lambda i: (i, 0)),
            pl.BlockSpec((n_cls, INPUT_DIM), lambda i: (0, 0)),
            pl.BlockSpec((n_cls,), lambda i: (0,)),
            pl.BlockSpec((n_box, INPUT_DIM), lambda i: (0, 0)),
            pl.BlockSpec((n_box,), lambda i: (0,)),
        ],
        out_specs=[
            pl.BlockSpec((ROW_BLOCK, n_cls), lambda i: (i, 0)),
            pl.BlockSpec((ROW_BLOCK, n_box), lambda i: (i, 0)),
        ],
        out_shape=[
            jax.ShapeDtypeStruct((N, n_cls), jnp.float32),
            jax.ShapeDtypeStruct((N, n_box), jnp.float32),
        ],
    )(x, W_cls, b_cls, W_bbox, b_bbox)
    return (scores, deltas)
